# baseline (speedup 1.0000x reference)
import jax
import jax.numpy as jnp
from jax import lax
from jax.experimental import pallas as pl
from jax.experimental.pallas import tpu as pltpu
from jax.experimental.pallas import tpu_sc as plsc

_TOKENS = 32768
_EXPERTS = 64
_NW = 32
_TPW = _TOKENS // _NW
_ROWS = _TOKENS // 2
_BR = 1024
_NB = _ROWS // _BR


def _tc_psum(x_ref, psum_ref):
    x = x_ref[...]
    xa = x[:, :_EXPERTS]
    xb = x[:, _EXPERTS:]

    def _psum(v):
        mx = jnp.max(v, axis=1, keepdims=True)
        e = jnp.exp(v - mx)
        s = jnp.sum(e, axis=1, keepdims=True)
        return jnp.sum(e / s, axis=0)

    psum_ref[...] = (_psum(xa) + _psum(xb)).reshape(1, 1, _EXPERTS)


def _sc_body(x_hbm, out_hbm, x_v, part_v):
    nc = 2
    wid = lax.axis_index("s") * nc + lax.axis_index("c")
    rows = _TPW // 2
    pltpu.sync_copy(x_hbm.at[pl.ds(wid * rows, rows)], x_v)

    ia = jnp.arange(16, dtype=jnp.int32)
    lt8 = ia < 8
    idx_t8 = jnp.full((16,), 8, dtype=jnp.int32)
    one = jnp.ones((16,), dtype=jnp.float32)
    zero = jnp.zeros((16,), dtype=jnp.float32)
    neg = jnp.full((16,), -jnp.inf, dtype=jnp.float32)

    def _sortd(v):
        return plsc.sort_key_val(v, v, descending=True)[0]

    def _mask_one(v0, v1, v2, v3, m0, m1, m2, m3):
        c01 = jnp.where(lt8, _sortd(v0), jnp.sort(v1))
        c23 = jnp.where(lt8, _sortd(v2), jnp.sort(v3))
        cf = jnp.where(lt8, _sortd(c01), jnp.sort(c23))
        sf = jnp.sort(cf)
        t8 = sf.at[idx_t8].get(mode="promise_in_bounds")
        m0 = jnp.maximum(m0, v0 - t8)
        m1 = jnp.maximum(m1, v1 - t8)
        m2 = jnp.maximum(m2, v2 - t8)
        m3 = jnp.maximum(m3, v3 - t8)
        return m0, m1, m2, m3

    init = (neg, neg, neg, neg, neg, neg, neg, neg)

    @plsc.parallel_loop(0, rows, unroll=4, carry=init)
    def body(r, carry):
        a0, a1, a2, a3, b0, b1, b2, b3 = carry
        v0 = x_v[r, pl.ds(0, 16)]
        v1 = x_v[r, pl.ds(16, 16)]
        v2 = x_v[r, pl.ds(32, 16)]
        v3 = x_v[r, pl.ds(48, 16)]
        w0 = x_v[r, pl.ds(64, 16)]
        w1 = x_v[r, pl.ds(80, 16)]
        w2 = x_v[r, pl.ds(96, 16)]
        w3 = x_v[r, pl.ds(112, 16)]
        a0, a1, a2, a3 = _mask_one(v0, v1, v2, v3, a0, a1, a2, a3)
        b0, b1, b2, b3 = _mask_one(w0, w1, w2, w3, b0, b1, b2, b3)
        return a0, a1, a2, a3, b0, b1, b2, b3

    a0, a1, a2, a3, b0, b1, b2, b3 = body
    part_v[pl.ds(0, 16)] = jnp.where(jnp.maximum(a0, b0) >= 0.0, one, zero)
    part_v[pl.ds(16, 16)] = jnp.where(jnp.maximum(a1, b1) >= 0.0, one, zero)
    part_v[pl.ds(32, 16)] = jnp.where(jnp.maximum(a2, b2) >= 0.0, one, zero)
    part_v[pl.ds(48, 16)] = jnp.where(jnp.maximum(a3, b3) >= 0.0, one, zero)
    pltpu.sync_copy(part_v, out_hbm.at[pl.ds(wid * _EXPERTS, _EXPERTS)])


def _tc_final(psum_ref, mask_ref, out_ref):
    psum = jnp.sum(psum_ref[...], axis=(0, 1))
    mask = jnp.max(mask_ref[...], axis=0)
    t = jnp.sum(psum) * jnp.float32(1.0 / _TOKENS)
    msum = jnp.sum(mask)
    out_ref[...] = jnp.full((1, 1), msum * t, jnp.float32)


def kernel(gate_logits):
    x2 = gate_logits.reshape(_ROWS, 2 * _EXPERTS)

    mesh = plsc.VectorSubcoreMesh(core_axis_name="c", subcore_axis_name="s")
    sc = pl.kernel(
        _sc_body,
        mesh=mesh,
        out_type=jax.ShapeDtypeStruct((_NW * _EXPERTS,), jnp.float32),
        scratch_types=[
            pltpu.VMEM((_TPW // 2, 2 * _EXPERTS), jnp.float32),
            pltpu.VMEM((_EXPERTS,), jnp.float32),
        ],
        compiler_params=pltpu.CompilerParams(needs_layout_passes=False),
    )
    masks = sc(x2).reshape(_NW, _EXPERTS)

    psums = pl.pallas_call(
        _tc_psum,
        grid=(_NB,),
        in_specs=[pl.BlockSpec((_BR, 2 * _EXPERTS), lambda i: (i, 0))],
        out_specs=pl.BlockSpec((1, 1, _EXPERTS), lambda i: (i, 0, 0)),
        out_shape=jax.ShapeDtypeStruct((_NB, 1, _EXPERTS), jnp.float32),
    )(x2)

    out = pl.pallas_call(
        _tc_final,
        out_shape=jax.ShapeDtypeStruct((1, 1), jnp.float32),
    )(psums, masks)
    return out[0, 0]


# speedup vs baseline: 1.3915x; 1.3915x over previous
import jax
import jax.numpy as jnp
from jax import lax
from jax.experimental import pallas as pl
from jax.experimental.pallas import tpu as pltpu
from jax.experimental.pallas import tpu_sc as plsc

_TOKENS = 32768
_EXPERTS = 64
_NW = 32
_TPW = _TOKENS // _NW
_CHUNK = 256


def _sc_body(x_hbm, out_hbm, xa_v, xb_v, part_v, sem_a, sem_b):
    nc = 2
    wid = lax.axis_index("s") * nc + lax.axis_index("c")
    base = wid * _TPW

    ia = jnp.arange(16, dtype=jnp.int32)
    lt8 = ia < 8
    idx_t8 = jnp.full((16,), 8, dtype=jnp.int32)
    idx_mx = jnp.full((16,), 15, dtype=jnp.int32)
    one = jnp.ones((16,), dtype=jnp.float32)
    zero = jnp.zeros((16,), dtype=jnp.float32)
    neg = jnp.full((16,), -jnp.inf, dtype=jnp.float32)
    two = jnp.float32(2.0)

    def _sortd(v):
        return plsc.sort_key_val(v, v, descending=True)[0]

    acc = (zero, zero, zero, zero, neg, neg, neg, neg)
    bufs = ((xa_v, sem_a), (xb_v, sem_b))
    pending = [
        pltpu.async_copy(x_hbm.at[pl.ds(base, _CHUNK)], xa_v, sem_a),
        pltpu.async_copy(x_hbm.at[pl.ds(base + _CHUNK, _CHUNK)], xb_v, sem_b),
    ]
    nchunks = _TPW // _CHUNK
    for c in range(nchunks):
        x_v, sem = bufs[c % 2]
        pending[c % 2].wait()

        @plsc.parallel_loop(0, _CHUNK, unroll=4, carry=acc)
        def body(t, carry):
            p0, p1, p2, p3, m0, m1, m2, m3 = carry
            v0 = x_v[t, pl.ds(0, 16)]
            v1 = x_v[t, pl.ds(16, 16)]
            v2 = x_v[t, pl.ds(32, 16)]
            v3 = x_v[t, pl.ds(48, 16)]

            c01 = jnp.where(lt8, _sortd(v0), jnp.sort(v1))
            c23 = jnp.where(lt8, _sortd(v2), jnp.sort(v3))
            cf = jnp.where(lt8, _sortd(c01), jnp.sort(c23))
            sf = jnp.sort(cf)
            t8 = sf.at[idx_t8].get(mode="promise_in_bounds")
            mx = sf.at[idx_mx].get(mode="promise_in_bounds")

            m0 = jnp.maximum(m0, v0 - t8)
            m1 = jnp.maximum(m1, v1 - t8)
            m2 = jnp.maximum(m2, v2 - t8)
            m3 = jnp.maximum(m3, v3 - t8)

            e0 = jnp.exp(v0 - mx)
            e1 = jnp.exp(v1 - mx)
            e2 = jnp.exp(v2 - mx)
            e3 = jnp.exp(v3 - mx)
            tot = (e0 + e1) + (e2 + e3)
            sv = jnp.broadcast_to(jnp.sum(tot), (16,))
            r = lax.bitcast_convert_type(
                jnp.int32(0x7EF311C3) - lax.bitcast_convert_type(sv, jnp.int32),
                jnp.float32,
            )
            r = r * (two - sv * r)
            inv = r * (two - sv * r)
            p0 = p0 + e0 * inv
            p1 = p1 + e1 * inv
            p2 = p2 + e2 * inv
            p3 = p3 + e3 * inv
            return p0, p1, p2, p3, m0, m1, m2, m3

        acc = body
        if c + 2 < nchunks:
            pending[c % 2] = pltpu.async_copy(
                x_hbm.at[pl.ds(base + (c + 2) * _CHUNK, _CHUNK)], x_v, sem
            )

    p0, p1, p2, p3, m0, m1, m2, m3 = acc
    part_v[pl.ds(0, 16)] = p0
    part_v[pl.ds(16, 16)] = p1
    part_v[pl.ds(32, 16)] = p2
    part_v[pl.ds(48, 16)] = p3
    part_v[pl.ds(64, 16)] = jnp.where(m0 >= 0.0, one, zero)
    part_v[pl.ds(80, 16)] = jnp.where(m1 >= 0.0, one, zero)
    part_v[pl.ds(96, 16)] = jnp.where(m2 >= 0.0, one, zero)
    part_v[pl.ds(112, 16)] = jnp.where(m3 >= 0.0, one, zero)
    pltpu.sync_copy(part_v, out_hbm.at[pl.ds(wid * 128, 128)])


def _tc_final(part_ref, out_ref):
    x = part_ref[...]
    psum = jnp.sum(x[:, :_EXPERTS], axis=0)
    mask = jnp.max(x[:, _EXPERTS:], axis=0)
    t = jnp.sum(psum) * jnp.float32(1.0 / _TOKENS)
    msum = jnp.sum(mask)
    out_ref[...] = jnp.full((1, 1), msum * t, jnp.float32)


def kernel(gate_logits):
    mesh = plsc.VectorSubcoreMesh(core_axis_name="c", subcore_axis_name="s")
    sc = pl.kernel(
        _sc_body,
        mesh=mesh,
        out_type=jax.ShapeDtypeStruct((_NW * 128,), jnp.float32),
        scratch_types=[
            pltpu.VMEM((_CHUNK, _EXPERTS), jnp.float32),
            pltpu.VMEM((_CHUNK, _EXPERTS), jnp.float32),
            pltpu.VMEM((128,), jnp.float32),
            pltpu.SemaphoreType.DMA,
            pltpu.SemaphoreType.DMA,
        ],
        compiler_params=pltpu.CompilerParams(needs_layout_passes=False),
    )
    parts = sc(gate_logits).reshape(_NW, 128)

    out = pl.pallas_call(
        _tc_final,
        out_shape=jax.ShapeDtypeStruct((1, 1), jnp.float32),
    )(parts)
    return out[0, 0]


# speedup vs baseline: 1.6056x; 1.1539x over previous
import jax
import jax.numpy as jnp
from jax import lax
from jax.experimental import pallas as pl
from jax.experimental.pallas import tpu as pltpu
from jax.experimental.pallas import tpu_sc as plsc

_TOKENS = 32768
_EXPERTS = 64
_NW = 32
_TPW = _TOKENS // _NW
_BT = 4096
_NB = _TOKENS // _BT
_CHUNK = 256


def _tc_psum(x_ref, psum_ref):
    x = x_ref[...]
    mx = jnp.max(x, axis=1, keepdims=True)
    e = jnp.exp(x - mx)
    s = jnp.sum(e, axis=1, keepdims=True)
    p = e / s
    psum_ref[...] = jnp.sum(p, axis=0).reshape(1, 1, _EXPERTS)


def _sc_body(x_hbm, out_hbm, xa_v, xb_v, part_v, sem_a, sem_b):
    nc = 2
    wid = lax.axis_index("s") * nc + lax.axis_index("c")
    base = wid * _TPW

    ia = jnp.arange(16, dtype=jnp.int32)
    lt8 = ia < 8
    idx_t8 = jnp.full((16,), 8, dtype=jnp.int32)
    one = jnp.ones((16,), dtype=jnp.float32)
    zero = jnp.zeros((16,), dtype=jnp.float32)
    neg = jnp.full((16,), -jnp.inf, dtype=jnp.float32)

    def _sortd(v):
        return plsc.sort_key_val(v, v, descending=True)[0]

    acc = (neg, neg, neg, neg)
    bufs = ((xa_v, sem_a), (xb_v, sem_b))
    pending = [
        pltpu.async_copy(x_hbm.at[pl.ds(base, _CHUNK)], xa_v, sem_a),
        pltpu.async_copy(x_hbm.at[pl.ds(base + _CHUNK, _CHUNK)], xb_v, sem_b),
    ]
    nchunks = _TPW // _CHUNK
    for c in range(nchunks):
        x_v, sem = bufs[c % 2]
        pending[c % 2].wait()

        @plsc.parallel_loop(0, _CHUNK, unroll=4, carry=acc)
        def body(t, carry):
            m0, m1, m2, m3 = carry
            v0 = x_v[t, pl.ds(0, 16)]
            v1 = x_v[t, pl.ds(16, 16)]
            v2 = x_v[t, pl.ds(32, 16)]
            v3 = x_v[t, pl.ds(48, 16)]
            c01 = jnp.where(lt8, _sortd(v0), jnp.sort(v1))
            c23 = jnp.where(lt8, _sortd(v2), jnp.sort(v3))
            cf = jnp.where(lt8, _sortd(c01), jnp.sort(c23))
            sf = jnp.sort(cf)
            t8 = sf.at[idx_t8].get(mode="promise_in_bounds")
            m0 = jnp.maximum(m0, v0 - t8)
            m1 = jnp.maximum(m1, v1 - t8)
            m2 = jnp.maximum(m2, v2 - t8)
            m3 = jnp.maximum(m3, v3 - t8)
            return m0, m1, m2, m3

        acc = body
        if c + 2 < nchunks:
            pending[c % 2] = pltpu.async_copy(
                x_hbm.at[pl.ds(base + (c + 2) * _CHUNK, _CHUNK)], x_v, sem
            )

    m0, m1, m2, m3 = acc
    part_v[pl.ds(0, 16)] = jnp.where(m0 >= 0.0, one, zero)
    part_v[pl.ds(16, 16)] = jnp.where(m1 >= 0.0, one, zero)
    part_v[pl.ds(32, 16)] = jnp.where(m2 >= 0.0, one, zero)
    part_v[pl.ds(48, 16)] = jnp.where(m3 >= 0.0, one, zero)
    pltpu.sync_copy(part_v, out_hbm.at[wid])


def _tc_final(psum_ref, mask_ref, out_ref):
    psum = jnp.sum(psum_ref[...], axis=(0, 1))
    mask = jnp.max(mask_ref[...], axis=0)
    t = jnp.sum(psum) * jnp.float32(1.0 / _TOKENS)
    msum = jnp.sum(mask)
    out_ref[...] = jnp.full((1, 1), msum * t, jnp.float32)


def kernel(gate_logits):
    mesh = plsc.VectorSubcoreMesh(core_axis_name="c", subcore_axis_name="s")
    sc = pl.kernel(
        _sc_body,
        mesh=mesh,
        out_type=jax.ShapeDtypeStruct((_NW, _EXPERTS), jnp.float32),
        scratch_types=[
            pltpu.VMEM((_CHUNK, _EXPERTS), jnp.float32),
            pltpu.VMEM((_CHUNK, _EXPERTS), jnp.float32),
            pltpu.VMEM((_EXPERTS,), jnp.float32),
            pltpu.SemaphoreType.DMA,
            pltpu.SemaphoreType.DMA,
        ],
        compiler_params=pltpu.CompilerParams(needs_layout_passes=False),
    )
    masks = sc(gate_logits)

    psums = pl.pallas_call(
        _tc_psum,
        grid=(_NB,),
        in_specs=[pl.BlockSpec((_BT, _EXPERTS), lambda i: (i, 0))],
        out_specs=pl.BlockSpec((1, 1, _EXPERTS), lambda i: (i, 0, 0)),
        out_shape=jax.ShapeDtypeStruct((_NB, 1, _EXPERTS), jnp.float32),
    )(gate_logits)

    out = pl.pallas_call(
        _tc_final,
        out_shape=jax.ShapeDtypeStruct((1, 1), jnp.float32),
    )(psums, masks)
    return out[0, 0]


# speedup vs baseline: 1.6281x; 1.0140x over previous
import jax
import jax.numpy as jnp
from jax import lax
from jax.experimental import pallas as pl
from jax.experimental.pallas import tpu as pltpu
from jax.experimental.pallas import tpu_sc as plsc

_TOKENS = 32768
_EXPERTS = 64
_NW = 32
_TPW = _TOKENS // _NW
_BT = 8192
_NB = _TOKENS // _BT
_CHUNK = 256


def _tc_psum(x_ref, psum_ref):
    x = x_ref[...]
    mx = jnp.max(x, axis=1, keepdims=True)
    e = jnp.exp(x - mx)
    s = jnp.sum(e, axis=1, keepdims=True)
    p = e / s
    psum_ref[...] = jnp.sum(p, axis=0).reshape(1, 1, _EXPERTS)


def _sc_body(x_hbm, out_hbm, xa_v, xb_v, part_v, sem_a, sem_b):
    nc = 2
    wid = lax.axis_index("s") * nc + lax.axis_index("c")
    base = wid * _TPW

    ia = jnp.arange(16, dtype=jnp.int32)
    lt8 = ia < 8
    idx_t8 = jnp.full((16,), 8, dtype=jnp.int32)
    one = jnp.ones((16,), dtype=jnp.float32)
    zero = jnp.zeros((16,), dtype=jnp.float32)
    neg = jnp.full((16,), -jnp.inf, dtype=jnp.float32)

    def _sortd(v):
        return plsc.sort_key_val(v, v, descending=True)[0]

    acc = (neg, neg, neg, neg)
    bufs = ((xa_v, sem_a), (xb_v, sem_b))
    pending = [
        pltpu.async_copy(x_hbm.at[pl.ds(base, _CHUNK)], xa_v, sem_a),
        pltpu.async_copy(x_hbm.at[pl.ds(base + _CHUNK, _CHUNK)], xb_v, sem_b),
    ]
    nchunks = _TPW // _CHUNK
    for c in range(nchunks):
        x_v, sem = bufs[c % 2]
        pending[c % 2].wait()

        @plsc.parallel_loop(0, _CHUNK, unroll=4, carry=acc)
        def body(t, carry):
            m0, m1, m2, m3 = carry
            v0 = x_v[t, pl.ds(0, 16)]
            v1 = x_v[t, pl.ds(16, 16)]
            v2 = x_v[t, pl.ds(32, 16)]
            v3 = x_v[t, pl.ds(48, 16)]
            c01 = jnp.where(lt8, _sortd(v0), jnp.sort(v1))
            c23 = jnp.where(lt8, _sortd(v2), jnp.sort(v3))
            cf = jnp.where(lt8, _sortd(c01), jnp.sort(c23))
            sf = jnp.sort(cf)
            t8 = sf.at[idx_t8].get(mode="promise_in_bounds")
            m0 = jnp.maximum(m0, v0 - t8)
            m1 = jnp.maximum(m1, v1 - t8)
            m2 = jnp.maximum(m2, v2 - t8)
            m3 = jnp.maximum(m3, v3 - t8)
            return m0, m1, m2, m3

        acc = body
        if c + 2 < nchunks:
            pending[c % 2] = pltpu.async_copy(
                x_hbm.at[pl.ds(base + (c + 2) * _CHUNK, _CHUNK)], x_v, sem
            )

    m0, m1, m2, m3 = acc
    part_v[pl.ds(0, 16)] = jnp.where(m0 >= 0.0, one, zero)
    part_v[pl.ds(16, 16)] = jnp.where(m1 >= 0.0, one, zero)
    part_v[pl.ds(32, 16)] = jnp.where(m2 >= 0.0, one, zero)
    part_v[pl.ds(48, 16)] = jnp.where(m3 >= 0.0, one, zero)
    pltpu.sync_copy(part_v, out_hbm.at[wid])


def _tc_final(psum_ref, mask_ref, out_ref):
    psum = jnp.sum(psum_ref[...], axis=(0, 1))
    mask = jnp.max(mask_ref[...], axis=0)
    t = jnp.sum(psum) * jnp.float32(1.0 / _TOKENS)
    msum = jnp.sum(mask)
    out_ref[...] = jnp.full((1, 1), msum * t, jnp.float32)


def kernel(gate_logits):
    mesh = plsc.VectorSubcoreMesh(core_axis_name="c", subcore_axis_name="s")
    sc = pl.kernel(
        _sc_body,
        mesh=mesh,
        out_type=jax.ShapeDtypeStruct((_NW, _EXPERTS), jnp.float32),
        scratch_types=[
            pltpu.VMEM((_CHUNK, _EXPERTS), jnp.float32),
            pltpu.VMEM((_CHUNK, _EXPERTS), jnp.float32),
            pltpu.VMEM((_EXPERTS,), jnp.float32),
            pltpu.SemaphoreType.DMA,
            pltpu.SemaphoreType.DMA,
        ],
        compiler_params=pltpu.CompilerParams(needs_layout_passes=False),
    )
    masks = sc(gate_logits)

    psums = pl.pallas_call(
        _tc_psum,
        grid=(_NB,),
        in_specs=[pl.BlockSpec((_BT, _EXPERTS), lambda i: (i, 0))],
        out_specs=pl.BlockSpec((1, 1, _EXPERTS), lambda i: (i, 0, 0)),
        out_shape=jax.ShapeDtypeStruct((_NB, 1, _EXPERTS), jnp.float32),
    )(gate_logits)

    out = pl.pallas_call(
        _tc_final,
        out_shape=jax.ShapeDtypeStruct((1, 1), jnp.float32),
    )(psums, masks)
    return out[0, 0]
